# Initial kernel scaffold; baseline (speedup 1.0000x reference)
import jax
import jax.numpy as jnp
from jax import lax
from jax.experimental import pallas as pl
from jax.experimental.pallas import tpu as pltpu
from jax.experimental.pallas import tpu_sc as plsc

N_NODES = 10000
N_EDGES = 320000
NC, NS = 2, 16
NW = NC * NS
EPW = N_EDGES // NW
CHUNK = 80
NCHUNK = EPW // CHUNK
ROWS_PER_SUB = N_NODES // NS
ZROWS = 125
CNTW = 16

_mesh = plsc.VectorSubcoreMesh(core_axis_name="c", subcore_axis_name="s")


def _zero_vmem(ref, rows, width):
    lanes = width // 16

    def body(i, carry):
        ref[i // lanes, pl.ds((i % lanes) * 16, 16)] = jnp.zeros((16,), jnp.float32)
        return carry

    lax.fori_loop(0, rows * lanes, body, 0)


def _make_sc_agg(width, with_cnt):
    out_types = [jax.ShapeDtypeStruct((NC, N_NODES, width), jnp.float32)]
    scratch = [
        pltpu.VMEM((NCHUNK, CHUNK), jnp.int32),
        pltpu.VMEM((NCHUNK, CHUNK), jnp.int32),
        pltpu.VMEM((CHUNK, width), jnp.float32),
        pltpu.VMEM((ZROWS, width), jnp.float32),
        pltpu.VMEM_SHARED((N_NODES, width), jnp.float32),
    ]
    if with_cnt:
        out_types.append(jax.ShapeDtypeStruct((NC, N_NODES, CNTW), jnp.float32))
        scratch += [
            pltpu.VMEM((CHUNK, CNTW), jnp.float32),
            pltpu.VMEM((ROWS_PER_SUB, CNTW), jnp.float32),
            pltpu.VMEM_SHARED((N_NODES, CNTW), jnp.float32),
        ]

    def body(y_hbm, src_hbm, dst_hbm, *rest):
        if with_cnt:
            (out_hbm, cnt_hbm, src_v, dst_v, rows_v, zbuf, acc,
             ones_v, zcnt, cntacc) = rest
        else:
            out_hbm, src_v, dst_v, rows_v, zbuf, acc = rest
        c = lax.axis_index("c")
        s = lax.axis_index("s")
        w = s * NC + c

        _zero_vmem(zbuf, ZROWS, width)
        for k in range(ROWS_PER_SUB // ZROWS):
            pltpu.sync_copy(zbuf, acc.at[pl.ds(s * ROWS_PER_SUB + k * ZROWS, ZROWS)])
        if with_cnt:
            _zero_vmem(zcnt, ROWS_PER_SUB, CNTW)
            pltpu.sync_copy(zcnt, cntacc.at[pl.ds(s * ROWS_PER_SUB, ROWS_PER_SUB)])

            def ones_body(i, carry):
                ones_v[i, pl.ds(0, 16)] = jnp.ones((16,), jnp.float32)
                return carry

            lax.fori_loop(0, CHUNK, ones_body, 0)
        plsc.subcore_barrier()

        pltpu.sync_copy(src_hbm.at[w], src_v)
        pltpu.sync_copy(dst_hbm.at[w], dst_v)

        def step(j, carry):
            pltpu.sync_copy(y_hbm.at[src_v.at[j]], rows_v)
            pltpu.sync_copy(rows_v, acc.at[dst_v.at[j]], add=True)
            if with_cnt:
                pltpu.sync_copy(ones_v, cntacc.at[dst_v.at[j]], add=True)
            return carry

        lax.fori_loop(0, NCHUNK, step, 0)
        plsc.subcore_barrier()

        rs = pl.ds(s * ROWS_PER_SUB, ROWS_PER_SUB)
        pltpu.sync_copy(acc.at[rs], out_hbm.at[c].at[rs])
        if with_cnt:
            pltpu.sync_copy(cntacc.at[rs], cnt_hbm.at[c].at[rs])

    return pl.kernel(
        body,
        out_type=tuple(out_types) if with_cnt else out_types[0],
        mesh=_mesh,
        scratch_types=scratch,
    )


_sc_agg128_cnt = _make_sc_agg(128, with_cnt=True)
_sc_agg128 = _make_sc_agg(128, with_cnt=False)
_sc_agg64 = _make_sc_agg(64, with_cnt=False)


_BR = 1000


def _tc_in_body(x_ref, wl_ref, wr_ref, b_ref, y_ref, z_ref):
    x = x_ref[...]
    y_ref[...] = jnp.dot(x, wl_ref[...], preferred_element_type=jnp.float32)
    z_ref[...] = jnp.dot(x, wr_ref[...], preferred_element_type=jnp.float32) + b_ref[...]


def _tc_in(x, wl, wr, b):
    d, h = wl.shape
    grid = (N_NODES // _BR,)
    return pl.pallas_call(
        _tc_in_body,
        grid=grid,
        in_specs=[
            pl.BlockSpec((_BR, d), lambda i: (i, 0)),
            pl.BlockSpec((d, h), lambda i: (0, 0)),
            pl.BlockSpec((d, h), lambda i: (0, 0)),
            pl.BlockSpec((1, h), lambda i: (0, 0)),
        ],
        out_specs=[
            pl.BlockSpec((_BR, h), lambda i: (i, 0)),
            pl.BlockSpec((_BR, h), lambda i: (i, 0)),
        ],
        out_shape=[
            jax.ShapeDtypeStruct((N_NODES, h), jnp.float32),
            jax.ShapeDtypeStruct((N_NODES, h), jnp.float32),
        ],
    )(x, wl, wr, b.reshape(1, h))


def _tc_mid_body(agg_ref, cnt_ref, z_ref, wl_ref, wr_ref, b_ref, y_ref, z2_ref):
    agg = agg_ref[0] + agg_ref[1]
    cnt = cnt_ref[0, :, 0:1] + cnt_ref[1, :, 0:1]
    h = jnp.maximum(agg / jnp.maximum(cnt, 1.0) + z_ref[...], 0.0)
    y_ref[...] = jnp.dot(h, wl_ref[...], preferred_element_type=jnp.float32)
    z2_ref[...] = jnp.dot(h, wr_ref[...], preferred_element_type=jnp.float32) + b_ref[...]


def _tc_mid(agg, cnt, z, wl, wr, b):
    d, h = wl.shape
    grid = (N_NODES // _BR,)
    return pl.pallas_call(
        _tc_mid_body,
        grid=grid,
        in_specs=[
            pl.BlockSpec((NC, _BR, d), lambda i: (0, i, 0)),
            pl.BlockSpec((NC, _BR, CNTW), lambda i: (0, i, 0)),
            pl.BlockSpec((_BR, d), lambda i: (i, 0)),
            pl.BlockSpec((d, h), lambda i: (0, 0)),
            pl.BlockSpec((d, h), lambda i: (0, 0)),
            pl.BlockSpec((1, h), lambda i: (0, 0)),
        ],
        out_specs=[
            pl.BlockSpec((_BR, h), lambda i: (i, 0)),
            pl.BlockSpec((_BR, h), lambda i: (i, 0)),
        ],
        out_shape=[
            jax.ShapeDtypeStruct((N_NODES, h), jnp.float32),
            jax.ShapeDtypeStruct((N_NODES, h), jnp.float32),
        ],
    )(agg, cnt, z, wl, wr, b.reshape(1, h))


def _tc_out_body(agg_ref, cnt_ref, z_ref, o_ref):
    agg = agg_ref[0] + agg_ref[1]
    cnt = cnt_ref[0, :, 0:1] + cnt_ref[1, :, 0:1]
    o_ref[...] = agg / jnp.maximum(cnt, 1.0) + z_ref[...]


def _tc_out(agg, cnt, z):
    h = z.shape[1]
    grid = (N_NODES // _BR,)
    return pl.pallas_call(
        _tc_out_body,
        grid=grid,
        in_specs=[
            pl.BlockSpec((NC, _BR, h), lambda i: (0, i, 0)),
            pl.BlockSpec((NC, _BR, CNTW), lambda i: (0, i, 0)),
            pl.BlockSpec((_BR, h), lambda i: (i, 0)),
        ],
        out_specs=pl.BlockSpec((_BR, h), lambda i: (i, 0)),
        out_shape=jax.ShapeDtypeStruct((N_NODES, h), jnp.float32),
    )(agg, cnt, z)


@jax.jit
def kernel(x, edge_index, Wl1, Wr1, b1, Wl2, Wr2, b2, Wl3, Wr3, b3):
    src = edge_index[0].astype(jnp.int32).reshape(NW, NCHUNK, CHUNK)
    dst = edge_index[1].astype(jnp.int32).reshape(NW, NCHUNK, CHUNK)

    y1, z1 = _tc_in(x, Wl1, Wr1, b1)
    agg1, cnt = _sc_agg128_cnt(y1, src, dst)
    y2, z2 = _tc_mid(agg1, cnt, z1, Wl2, Wr2, b2)
    agg2 = _sc_agg128(y2, src, dst)
    y3, z3 = _tc_mid(agg2, cnt, z2, Wl3, Wr3, b3)
    agg3 = _sc_agg64(y3, src, dst)
    return _tc_out(agg3, cnt, z3)


# speedup vs baseline: 6.0738x; 6.0738x over previous
import functools

import jax
import jax.numpy as jnp
from jax import lax
from jax.experimental import pallas as pl
from jax.experimental.pallas import tpu as pltpu
from jax.experimental.pallas import tpu_sc as plsc

N_NODES = 10000
N_EDGES = 320000
NC, NS = 2, 16
NW = NC * NS
EPW = N_EDGES // NW
CHUNK = 80
NCHUNK = EPW // CHUNK
NPAD = 10112
ROWS_PER_SUB = NPAD // NS
ZROWS = 128
CNTW = 16

_mesh = plsc.VectorSubcoreMesh(core_axis_name="c", subcore_axis_name="s")


def _zero_vmem(ref, rows, width):
    lanes = width // 16

    def body(i, carry):
        ref[i // lanes, pl.ds((i % lanes) * 16, 16)] = jnp.zeros((16,), jnp.float32)
        return carry

    lax.fori_loop(0, rows * lanes, body, 0)


def _clear_rows(zbuf, shared, s):
    base = s * ROWS_PER_SUB
    for k in range(ROWS_PER_SUB // ZROWS):
        pltpu.sync_copy(zbuf, shared.at[pl.ds(base + k * ZROWS, ZROWS)])
    rem = ROWS_PER_SUB % ZROWS
    if rem:
        pltpu.sync_copy(zbuf.at[pl.ds(0, rem)],
                        shared.at[pl.ds(base + (ROWS_PER_SUB // ZROWS) * ZROWS, rem)])


def _make_sc_agg(width):
    scratch = [
        pltpu.VMEM((NCHUNK, CHUNK), jnp.int32),
        pltpu.VMEM((NCHUNK, CHUNK), jnp.int32),
        pltpu.VMEM((CHUNK, width), jnp.float32),
        pltpu.VMEM((ZROWS, width), jnp.float32),
        pltpu.VMEM_SHARED((NPAD, width), jnp.float32),
    ]

    def body(y_hbm, src_hbm, dst_hbm, out_hbm, src_v, dst_v, rows_v, zbuf, acc):
        c = lax.axis_index("c")
        s = lax.axis_index("s")
        w = s * NC + c

        _zero_vmem(zbuf, ZROWS, width)
        _clear_rows(zbuf, acc, s)
        plsc.subcore_barrier()

        pltpu.sync_copy(src_hbm.at[w], src_v)
        pltpu.sync_copy(dst_hbm.at[w], dst_v)

        def step(j, carry):
            pltpu.sync_copy(y_hbm.at[src_v.at[j]], rows_v)
            pltpu.sync_copy(rows_v, acc.at[dst_v.at[j]], add=True)
            return carry

        lax.fori_loop(0, NCHUNK, step, 0)
        plsc.subcore_barrier()

        rs = pl.ds(s * ROWS_PER_SUB, ROWS_PER_SUB)
        pltpu.sync_copy(acc.at[rs], out_hbm.at[c].at[rs])

    return pl.kernel(
        body,
        out_type=jax.ShapeDtypeStruct((NC, NPAD, width), jnp.float32),
        mesh=_mesh,
        scratch_types=scratch,
        compiler_params=pltpu.CompilerParams(use_tc_tiling_on_sc=False),
    )


def _sc_cnt_body(dst_hbm, cnt_hbm, dst_v, ones_v, zbuf, cntacc):
    c = lax.axis_index("c")
    s = lax.axis_index("s")
    w = s * NC + c

    _zero_vmem(zbuf, ZROWS, CNTW)
    _clear_rows(zbuf, cntacc, s)

    def ones_body(i, carry):
        ones_v[i, pl.ds(0, 16)] = jnp.ones((16,), jnp.float32)
        return carry

    lax.fori_loop(0, CHUNK, ones_body, 0)
    plsc.subcore_barrier()

    pltpu.sync_copy(dst_hbm.at[w], dst_v)

    def step(j, carry):
        pltpu.sync_copy(ones_v, cntacc.at[dst_v.at[j]], add=True)
        return carry

    lax.fori_loop(0, NCHUNK, step, 0)
    plsc.subcore_barrier()

    rs = pl.ds(s * ROWS_PER_SUB, ROWS_PER_SUB)
    pltpu.sync_copy(cntacc.at[rs], cnt_hbm.at[c].at[rs])


_sc_cnt = pl.kernel(
    _sc_cnt_body,
    out_type=jax.ShapeDtypeStruct((NC, NPAD, CNTW), jnp.float32),
    mesh=_mesh,
    scratch_types=[
        pltpu.VMEM((NCHUNK, CHUNK), jnp.int32),
        pltpu.VMEM((CHUNK, CNTW), jnp.float32),
        pltpu.VMEM((ZROWS, CNTW), jnp.float32),
        pltpu.VMEM_SHARED((NPAD, CNTW), jnp.float32),
    ],
    compiler_params=pltpu.CompilerParams(use_tc_tiling_on_sc=False),
)

_sc_agg64 = _make_sc_agg(64)


_BR = 1000
AW = 64


def _tc_in_body(x_ref, wl_ref, wr_ref, b_ref, *out_refs):
    x = x_ref[...]
    y = jnp.dot(x, wl_ref[...], preferred_element_type=jnp.float32)
    for p, yr in enumerate(out_refs[:-1]):
        yr[...] = y[:, p * AW:(p + 1) * AW]
    out_refs[-1][...] = (
        jnp.dot(x, wr_ref[...], preferred_element_type=jnp.float32) + b_ref[...])


def _tc_in(x, wl, wr, b):
    d, h = wl.shape
    parts = h // AW
    grid = (N_NODES // _BR,)
    return pl.pallas_call(
        _tc_in_body,
        grid=grid,
        in_specs=[
            pl.BlockSpec((_BR, d), lambda i: (i, 0)),
            pl.BlockSpec((d, h), lambda i: (0, 0)),
            pl.BlockSpec((d, h), lambda i: (0, 0)),
            pl.BlockSpec((1, h), lambda i: (0, 0)),
        ],
        out_specs=[pl.BlockSpec((_BR, AW), lambda i: (i, 0))] * parts
                  + [pl.BlockSpec((_BR, h), lambda i: (i, 0))],
        out_shape=[jax.ShapeDtypeStruct((N_NODES, AW), jnp.float32)] * parts
                  + [jax.ShapeDtypeStruct((N_NODES, h), jnp.float32)],
    )(x, wl, wr, b.reshape(1, h))


def _mean_from_parts(agg_refs, cnt_ref):
    agg = jnp.concatenate([a[0] + a[1] for a in agg_refs], axis=1)
    cnt = cnt_ref[0, :, 0:1] + cnt_ref[1, :, 0:1]
    return agg / jnp.maximum(cnt, 1.0)


def _tc_mid_body(nparts, *refs):
    agg_refs = refs[:nparts]
    cnt_ref, z_ref, wl_ref, wr_ref, b_ref = refs[nparts:nparts + 5]
    out_refs = refs[nparts + 5:]
    h = jnp.maximum(_mean_from_parts(agg_refs, cnt_ref) + z_ref[...], 0.0)
    y = jnp.dot(h, wl_ref[...], preferred_element_type=jnp.float32)
    for p, yr in enumerate(out_refs[:-1]):
        yr[...] = y[:, p * AW:(p + 1) * AW]
    out_refs[-1][...] = (
        jnp.dot(h, wr_ref[...], preferred_element_type=jnp.float32) + b_ref[...])


def _tc_mid(agg_parts, cnt, z, wl, wr, b):
    d, h = wl.shape
    nparts = len(agg_parts)
    oparts = h // AW
    grid = (N_NODES // _BR,)
    return pl.pallas_call(
        functools.partial(_tc_mid_body, nparts),
        grid=grid,
        in_specs=[pl.BlockSpec((NC, _BR, AW), lambda i: (0, i, 0))] * nparts + [
            pl.BlockSpec((NC, _BR, CNTW), lambda i: (0, i, 0)),
            pl.BlockSpec((_BR, d), lambda i: (i, 0)),
            pl.BlockSpec((d, h), lambda i: (0, 0)),
            pl.BlockSpec((d, h), lambda i: (0, 0)),
            pl.BlockSpec((1, h), lambda i: (0, 0)),
        ],
        out_specs=[pl.BlockSpec((_BR, AW), lambda i: (i, 0))] * oparts
                  + [pl.BlockSpec((_BR, h), lambda i: (i, 0))],
        out_shape=[jax.ShapeDtypeStruct((N_NODES, AW), jnp.float32)] * oparts
                  + [jax.ShapeDtypeStruct((N_NODES, h), jnp.float32)],
    )(*agg_parts, cnt, z, wl, wr, b.reshape(1, h))


def _tc_out_body(agg_ref, cnt_ref, z_ref, o_ref):
    o_ref[...] = _mean_from_parts([agg_ref], cnt_ref) + z_ref[...]


def _tc_out(agg, cnt, z):
    h = z.shape[1]
    grid = (N_NODES // _BR,)
    return pl.pallas_call(
        _tc_out_body,
        grid=grid,
        in_specs=[
            pl.BlockSpec((NC, _BR, h), lambda i: (0, i, 0)),
            pl.BlockSpec((NC, _BR, CNTW), lambda i: (0, i, 0)),
            pl.BlockSpec((_BR, h), lambda i: (i, 0)),
        ],
        out_specs=pl.BlockSpec((_BR, h), lambda i: (i, 0)),
        out_shape=jax.ShapeDtypeStruct((N_NODES, h), jnp.float32),
    )(agg, cnt, z)


@jax.jit
def kernel(x, edge_index, Wl1, Wr1, b1, Wl2, Wr2, b2, Wl3, Wr3, b3):
    src = edge_index[0].astype(jnp.int32).reshape(NW, NCHUNK, CHUNK)
    dst = edge_index[1].astype(jnp.int32).reshape(NW, NCHUNK, CHUNK)

    cnt = _sc_cnt(dst)
    *y1, z1 = _tc_in(x, Wl1, Wr1, b1)
    agg1 = [_sc_agg64(yp, src, dst) for yp in y1]
    *y2, z2 = _tc_mid(agg1, cnt, z1, Wl2, Wr2, b2)
    agg2 = [_sc_agg64(yp, src, dst) for yp in y2]
    y3, z3 = _tc_mid(agg2, cnt, z2, Wl3, Wr3, b3)
    agg3 = _sc_agg64(y3, src, dst)
    return _tc_out(agg3, cnt, z3)


# speedup vs baseline: 6.8440x; 1.1268x over previous
import functools

import jax
import jax.numpy as jnp
from jax import lax
from jax.experimental import pallas as pl
from jax.experimental.pallas import tpu as pltpu
from jax.experimental.pallas import tpu_sc as plsc

N_NODES = 10000
N_EDGES = 320000
NC, NS = 2, 16
NW = NC * NS
EPW = N_EDGES // NW
CHUNK = 40
NCHUNK = EPW // CHUNK
NPAD = 10112
ROWS_PER_SUB = NPAD // NS
ZROWS = 128
CNTW = 16

_mesh = plsc.VectorSubcoreMesh(core_axis_name="c", subcore_axis_name="s")


def _zero_vmem(ref, rows, width):
    lanes = width // 16

    def body(i, carry):
        ref[i // lanes, pl.ds((i % lanes) * 16, 16)] = jnp.zeros((16,), jnp.float32)
        return carry

    lax.fori_loop(0, rows * lanes, body, 0)


def _clear_rows(zbuf, shared, s):
    base = s * ROWS_PER_SUB
    for k in range(ROWS_PER_SUB // ZROWS):
        pltpu.sync_copy(zbuf, shared.at[pl.ds(base + k * ZROWS, ZROWS)])
    rem = ROWS_PER_SUB % ZROWS
    if rem:
        pltpu.sync_copy(zbuf.at[pl.ds(0, rem)],
                        shared.at[pl.ds(base + (ROWS_PER_SUB // ZROWS) * ZROWS, rem)])


def _make_sc_agg(width):
    scratch = [
        pltpu.VMEM((NCHUNK, CHUNK), jnp.int32),
        pltpu.VMEM((NCHUNK, CHUNK), jnp.int32),
        pltpu.VMEM((CHUNK, width), jnp.float32),
        pltpu.VMEM((CHUNK, width), jnp.float32),
        pltpu.VMEM((ZROWS, width), jnp.float32),
        pltpu.VMEM_SHARED((NPAD, width), jnp.float32),
        pltpu.SemaphoreType.DMA,
        pltpu.SemaphoreType.DMA,
        pltpu.SemaphoreType.DMA,
        pltpu.SemaphoreType.DMA,
    ]

    def body(y_hbm, src_hbm, dst_hbm, out_hbm, src_v, dst_v, r0, r1, zbuf, acc,
             g0, g1, s0, s1):
        c = lax.axis_index("c")
        s = lax.axis_index("s")
        w = s * NC + c

        _zero_vmem(zbuf, ZROWS, width)
        _clear_rows(zbuf, acc, s)
        plsc.subcore_barrier()

        pltpu.sync_copy(src_hbm.at[w], src_v)
        pltpu.sync_copy(dst_hbm.at[w], dst_v)

        pltpu.async_copy(y_hbm.at[src_v.at[0]], r0, g0)
        pltpu.async_copy(y_hbm.at[src_v.at[1]], r1, g1)

        def step(t, carry):
            j = 2 * t
            pltpu.make_async_copy(y_hbm.at[src_v.at[j]], r0, g0).wait()
            pltpu.async_copy(r0, acc.at[dst_v.at[j]], s0, add=True)
            pltpu.make_async_copy(y_hbm.at[src_v.at[j + 1]], r1, g1).wait()
            pltpu.async_copy(r1, acc.at[dst_v.at[j + 1]], s1, add=True)
            pltpu.make_async_copy(r0, acc.at[dst_v.at[j]], s0).wait()
            pltpu.async_copy(y_hbm.at[src_v.at[j + 2]], r0, g0)
            pltpu.make_async_copy(r1, acc.at[dst_v.at[j + 1]], s1).wait()
            pltpu.async_copy(y_hbm.at[src_v.at[j + 3]], r1, g1)
            return carry

        lax.fori_loop(0, NCHUNK // 2 - 1, step, 0)

        jl = NCHUNK - 2
        pltpu.make_async_copy(y_hbm.at[src_v.at[jl]], r0, g0).wait()
        pltpu.async_copy(r0, acc.at[dst_v.at[jl]], s0, add=True)
        pltpu.make_async_copy(y_hbm.at[src_v.at[jl + 1]], r1, g1).wait()
        pltpu.async_copy(r1, acc.at[dst_v.at[jl + 1]], s1, add=True)
        pltpu.make_async_copy(r0, acc.at[dst_v.at[jl]], s0).wait()
        pltpu.make_async_copy(r1, acc.at[dst_v.at[jl + 1]], s1).wait()
        plsc.subcore_barrier()

        rs = pl.ds(s * ROWS_PER_SUB, ROWS_PER_SUB)
        pltpu.sync_copy(acc.at[rs], out_hbm.at[c].at[rs])

    return pl.kernel(
        body,
        out_type=jax.ShapeDtypeStruct((NC, NPAD, width), jnp.float32),
        mesh=_mesh,
        scratch_types=scratch,
        compiler_params=pltpu.CompilerParams(use_tc_tiling_on_sc=False),
    )


_CNT_GROUP = 10


def _sc_cnt_body(dst_hbm, cnt_hbm, dst_v, ones_v, zbuf, cntacc, sem):
    c = lax.axis_index("c")
    s = lax.axis_index("s")
    w = s * NC + c

    _zero_vmem(zbuf, ZROWS, CNTW)
    _clear_rows(zbuf, cntacc, s)

    def ones_body(i, carry):
        ones_v[i, pl.ds(0, 16)] = jnp.ones((16,), jnp.float32)
        return carry

    lax.fori_loop(0, CHUNK, ones_body, 0)
    plsc.subcore_barrier()

    pltpu.sync_copy(dst_hbm.at[w], dst_v)

    def step(t, carry):
        j = t * _CNT_GROUP
        for k in range(_CNT_GROUP):
            pltpu.async_copy(ones_v, cntacc.at[dst_v.at[j + k]], sem, add=True)
        for k in range(_CNT_GROUP):
            pltpu.make_async_copy(ones_v, cntacc.at[dst_v.at[j + k]], sem).wait()
        return carry

    lax.fori_loop(0, NCHUNK // _CNT_GROUP, step, 0)
    plsc.subcore_barrier()

    rs = pl.ds(s * ROWS_PER_SUB, ROWS_PER_SUB)
    pltpu.sync_copy(cntacc.at[rs], cnt_hbm.at[c].at[rs])


_sc_cnt = pl.kernel(
    _sc_cnt_body,
    out_type=jax.ShapeDtypeStruct((NC, NPAD, CNTW), jnp.float32),
    mesh=_mesh,
    scratch_types=[
        pltpu.VMEM((NCHUNK, CHUNK), jnp.int32),
        pltpu.VMEM((CHUNK, CNTW), jnp.float32),
        pltpu.VMEM((ZROWS, CNTW), jnp.float32),
        pltpu.VMEM_SHARED((NPAD, CNTW), jnp.float32),
        pltpu.SemaphoreType.DMA,
    ],
    compiler_params=pltpu.CompilerParams(use_tc_tiling_on_sc=False),
)

_sc_agg64 = _make_sc_agg(64)


_BR = 1000
AW = 64


def _tc_in_body(x_ref, wl_ref, wr_ref, b_ref, *out_refs):
    x = x_ref[...]
    y = jnp.dot(x, wl_ref[...], preferred_element_type=jnp.float32)
    for p, yr in enumerate(out_refs[:-1]):
        yr[...] = y[:, p * AW:(p + 1) * AW]
    out_refs[-1][...] = (
        jnp.dot(x, wr_ref[...], preferred_element_type=jnp.float32) + b_ref[...])


def _tc_in(x, wl, wr, b):
    d, h = wl.shape
    parts = h // AW
    grid = (N_NODES // _BR,)
    return pl.pallas_call(
        _tc_in_body,
        grid=grid,
        in_specs=[
            pl.BlockSpec((_BR, d), lambda i: (i, 0)),
            pl.BlockSpec((d, h), lambda i: (0, 0)),
            pl.BlockSpec((d, h), lambda i: (0, 0)),
            pl.BlockSpec((1, h), lambda i: (0, 0)),
        ],
        out_specs=[pl.BlockSpec((_BR, AW), lambda i: (i, 0))] * parts
                  + [pl.BlockSpec((_BR, h), lambda i: (i, 0))],
        out_shape=[jax.ShapeDtypeStruct((N_NODES, AW), jnp.float32)] * parts
                  + [jax.ShapeDtypeStruct((N_NODES, h), jnp.float32)],
    )(x, wl, wr, b.reshape(1, h))


def _mean_from_parts(agg_refs, cnt_ref):
    agg = jnp.concatenate([a[0] + a[1] for a in agg_refs], axis=1)
    cnt = cnt_ref[0, :, 0:1] + cnt_ref[1, :, 0:1]
    return agg / jnp.maximum(cnt, 1.0)


def _tc_mid_body(nparts, *refs):
    agg_refs = refs[:nparts]
    cnt_ref, z_ref, wl_ref, wr_ref, b_ref = refs[nparts:nparts + 5]
    out_refs = refs[nparts + 5:]
    h = jnp.maximum(_mean_from_parts(agg_refs, cnt_ref) + z_ref[...], 0.0)
    y = jnp.dot(h, wl_ref[...], preferred_element_type=jnp.float32)
    for p, yr in enumerate(out_refs[:-1]):
        yr[...] = y[:, p * AW:(p + 1) * AW]
    out_refs[-1][...] = (
        jnp.dot(h, wr_ref[...], preferred_element_type=jnp.float32) + b_ref[...])


def _tc_mid(agg_parts, cnt, z, wl, wr, b):
    d, h = wl.shape
    nparts = len(agg_parts)
    oparts = h // AW
    grid = (N_NODES // _BR,)
    return pl.pallas_call(
        functools.partial(_tc_mid_body, nparts),
        grid=grid,
        in_specs=[pl.BlockSpec((NC, _BR, AW), lambda i: (0, i, 0))] * nparts + [
            pl.BlockSpec((NC, _BR, CNTW), lambda i: (0, i, 0)),
            pl.BlockSpec((_BR, d), lambda i: (i, 0)),
            pl.BlockSpec((d, h), lambda i: (0, 0)),
            pl.BlockSpec((d, h), lambda i: (0, 0)),
            pl.BlockSpec((1, h), lambda i: (0, 0)),
        ],
        out_specs=[pl.BlockSpec((_BR, AW), lambda i: (i, 0))] * oparts
                  + [pl.BlockSpec((_BR, h), lambda i: (i, 0))],
        out_shape=[jax.ShapeDtypeStruct((N_NODES, AW), jnp.float32)] * oparts
                  + [jax.ShapeDtypeStruct((N_NODES, h), jnp.float32)],
    )(*agg_parts, cnt, z, wl, wr, b.reshape(1, h))


def _tc_out_body(agg_ref, cnt_ref, z_ref, o_ref):
    o_ref[...] = _mean_from_parts([agg_ref], cnt_ref) + z_ref[...]


def _tc_out(agg, cnt, z):
    h = z.shape[1]
    grid = (N_NODES // _BR,)
    return pl.pallas_call(
        _tc_out_body,
        grid=grid,
        in_specs=[
            pl.BlockSpec((NC, _BR, h), lambda i: (0, i, 0)),
            pl.BlockSpec((NC, _BR, CNTW), lambda i: (0, i, 0)),
            pl.BlockSpec((_BR, h), lambda i: (i, 0)),
        ],
        out_specs=pl.BlockSpec((_BR, h), lambda i: (i, 0)),
        out_shape=jax.ShapeDtypeStruct((N_NODES, h), jnp.float32),
    )(agg, cnt, z)


@jax.jit
def kernel(x, edge_index, Wl1, Wr1, b1, Wl2, Wr2, b2, Wl3, Wr3, b3):
    src = edge_index[0].astype(jnp.int32).reshape(NW, NCHUNK, CHUNK)
    dst = edge_index[1].astype(jnp.int32).reshape(NW, NCHUNK, CHUNK)

    cnt = _sc_cnt(dst)
    *y1, z1 = _tc_in(x, Wl1, Wr1, b1)
    agg1 = [_sc_agg64(yp, src, dst) for yp in y1]
    *y2, z2 = _tc_mid(agg1, cnt, z1, Wl2, Wr2, b2)
    agg2 = [_sc_agg64(yp, src, dst) for yp in y2]
    y3, z3 = _tc_mid(agg2, cnt, z2, Wl3, Wr3, b3)
    agg3 = _sc_agg64(y3, src, dst)
    return _tc_out(agg3, cnt, z3)


# speedup vs baseline: 8.4843x; 1.2397x over previous
import functools

import jax
import jax.numpy as jnp
from jax import lax
from jax.experimental import pallas as pl
from jax.experimental.pallas import tpu as pltpu
from jax.experimental.pallas import tpu_sc as plsc

N_NODES = 10000
N_EDGES = 320000
NC, NS = 2, 16
NW = NC * NS
EPW = N_EDGES // NW
CHUNK = 80
NCHUNK = EPW // CHUNK
NPAD = 10112
ROWS_PER_SUB = NPAD // NS
ZROWS = 128
CNTW = 16

_mesh = plsc.VectorSubcoreMesh(core_axis_name="c", subcore_axis_name="s")


def _zero_vmem(ref, rows, width):
    lanes = width // 16

    def body(i, carry):
        ref[i // lanes, pl.ds((i % lanes) * 16, 16)] = jnp.zeros((16,), jnp.float32)
        return carry

    lax.fori_loop(0, rows * lanes, body, 0)


def _clear_rows(zbuf, shared, s):
    base = s * ROWS_PER_SUB
    for k in range(ROWS_PER_SUB // ZROWS):
        pltpu.sync_copy(zbuf, shared.at[pl.ds(base + k * ZROWS, ZROWS)])
    rem = ROWS_PER_SUB % ZROWS
    if rem:
        pltpu.sync_copy(zbuf.at[pl.ds(0, rem)],
                        shared.at[pl.ds(base + (ROWS_PER_SUB // ZROWS) * ZROWS, rem)])


def _make_sc_agg(width):
    scratch = [
        pltpu.VMEM((NCHUNK, CHUNK), jnp.int32),
        pltpu.VMEM((NCHUNK, CHUNK), jnp.int32),
        pltpu.VMEM((CHUNK, width), jnp.float32),
        pltpu.VMEM((CHUNK, width), jnp.float32),
        pltpu.VMEM((ZROWS, width), jnp.float32),
        pltpu.VMEM_SHARED((NPAD, width), jnp.float32),
        pltpu.SemaphoreType.DMA,
        pltpu.SemaphoreType.DMA,
        pltpu.SemaphoreType.DMA,
        pltpu.SemaphoreType.DMA,
    ]

    def body(y_hbm, src_hbm, dst_hbm, out_hbm, src_v, dst_v, r0, r1, zbuf, acc,
             g0, g1, s0, s1):
        c = lax.axis_index("c")
        s = lax.axis_index("s")
        w = s * NC + c

        _zero_vmem(zbuf, ZROWS, width)
        _clear_rows(zbuf, acc, s)
        plsc.subcore_barrier()

        pltpu.sync_copy(src_hbm.at[w], src_v)
        pltpu.sync_copy(dst_hbm.at[w], dst_v)

        pltpu.async_copy(y_hbm.at[src_v.at[0]], r0, g0)
        pltpu.async_copy(y_hbm.at[src_v.at[1]], r1, g1)

        def step(t, carry):
            j = 2 * t
            pltpu.make_async_copy(y_hbm.at[src_v.at[j]], r0, g0).wait()
            pltpu.async_copy(r0, acc.at[dst_v.at[j]], s0, add=True)
            pltpu.make_async_copy(y_hbm.at[src_v.at[j + 1]], r1, g1).wait()
            pltpu.async_copy(r1, acc.at[dst_v.at[j + 1]], s1, add=True)
            pltpu.make_async_copy(r0, acc.at[dst_v.at[j]], s0).wait()
            pltpu.async_copy(y_hbm.at[src_v.at[j + 2]], r0, g0)
            pltpu.make_async_copy(r1, acc.at[dst_v.at[j + 1]], s1).wait()

            @pl.when(j + 3 < NCHUNK)
            def _():
                pltpu.async_copy(y_hbm.at[src_v.at[j + 3]], r1, g1)

            return carry

        lax.fori_loop(0, (NCHUNK - 1) // 2, step, 0)

        if NCHUNK % 2:
            jl = NCHUNK - 1
            pltpu.make_async_copy(y_hbm.at[src_v.at[jl]], r0, g0).wait()
            pltpu.async_copy(r0, acc.at[dst_v.at[jl]], s0, add=True)
            pltpu.make_async_copy(r0, acc.at[dst_v.at[jl]], s0).wait()
        else:
            jl = NCHUNK - 2
            pltpu.make_async_copy(y_hbm.at[src_v.at[jl]], r0, g0).wait()
            pltpu.async_copy(r0, acc.at[dst_v.at[jl]], s0, add=True)
            pltpu.make_async_copy(y_hbm.at[src_v.at[jl + 1]], r1, g1).wait()
            pltpu.async_copy(r1, acc.at[dst_v.at[jl + 1]], s1, add=True)
            pltpu.make_async_copy(r0, acc.at[dst_v.at[jl]], s0).wait()
            pltpu.make_async_copy(r1, acc.at[dst_v.at[jl + 1]], s1).wait()
        plsc.subcore_barrier()

        rs = pl.ds(s * ROWS_PER_SUB, ROWS_PER_SUB)
        pltpu.sync_copy(acc.at[rs], out_hbm.at[c].at[rs])

    return pl.kernel(
        body,
        out_type=jax.ShapeDtypeStruct((NC, NPAD, width), jnp.float32),
        mesh=_mesh,
        scratch_types=scratch,
        compiler_params=pltpu.CompilerParams(use_tc_tiling_on_sc=False),
    )


_CNT_GROUP = 5


def _sc_cnt_body(dst_hbm, cnt_hbm, dst_v, ones_v, zbuf, cntacc, sem):
    c = lax.axis_index("c")
    s = lax.axis_index("s")
    w = s * NC + c

    _zero_vmem(zbuf, ZROWS, CNTW)
    _clear_rows(zbuf, cntacc, s)

    def ones_body(i, carry):
        ones_v[i, pl.ds(0, 16)] = jnp.ones((16,), jnp.float32)
        return carry

    lax.fori_loop(0, CHUNK, ones_body, 0)
    plsc.subcore_barrier()

    pltpu.sync_copy(dst_hbm.at[w], dst_v)

    def step(t, carry):
        j = t * _CNT_GROUP
        for k in range(_CNT_GROUP):
            pltpu.async_copy(ones_v, cntacc.at[dst_v.at[j + k]], sem, add=True)
        for k in range(_CNT_GROUP):
            pltpu.make_async_copy(ones_v, cntacc.at[dst_v.at[j + k]], sem).wait()
        return carry

    lax.fori_loop(0, NCHUNK // _CNT_GROUP, step, 0)
    plsc.subcore_barrier()

    rs = pl.ds(s * ROWS_PER_SUB, ROWS_PER_SUB)
    pltpu.sync_copy(cntacc.at[rs], cnt_hbm.at[c].at[rs])


_sc_cnt = pl.kernel(
    _sc_cnt_body,
    out_type=jax.ShapeDtypeStruct((NC, NPAD, CNTW), jnp.float32),
    mesh=_mesh,
    scratch_types=[
        pltpu.VMEM((NCHUNK, CHUNK), jnp.int32),
        pltpu.VMEM((CHUNK, CNTW), jnp.float32),
        pltpu.VMEM((ZROWS, CNTW), jnp.float32),
        pltpu.VMEM_SHARED((NPAD, CNTW), jnp.float32),
        pltpu.SemaphoreType.DMA,
    ],
    compiler_params=pltpu.CompilerParams(use_tc_tiling_on_sc=False),
)

_sc_agg64 = _make_sc_agg(64)


_BR = 1000
AW = 64


def _tc_in_body(x_ref, wl_ref, wr_ref, b_ref, *out_refs):
    x = x_ref[...]
    y = jnp.dot(x, wl_ref[...], preferred_element_type=jnp.float32)
    for p, yr in enumerate(out_refs[:-1]):
        yr[...] = y[:, p * AW:(p + 1) * AW]
    out_refs[-1][...] = (
        jnp.dot(x, wr_ref[...], preferred_element_type=jnp.float32) + b_ref[...])


def _tc_in(x, wl, wr, b):
    d, h = wl.shape
    parts = h // AW
    grid = (N_NODES // _BR,)
    return pl.pallas_call(
        _tc_in_body,
        grid=grid,
        in_specs=[
            pl.BlockSpec((_BR, d), lambda i: (i, 0)),
            pl.BlockSpec((d, h), lambda i: (0, 0)),
            pl.BlockSpec((d, h), lambda i: (0, 0)),
            pl.BlockSpec((1, h), lambda i: (0, 0)),
        ],
        out_specs=[pl.BlockSpec((_BR, AW), lambda i: (i, 0))] * parts
                  + [pl.BlockSpec((_BR, h), lambda i: (i, 0))],
        out_shape=[jax.ShapeDtypeStruct((N_NODES, AW), jnp.float32)] * parts
                  + [jax.ShapeDtypeStruct((N_NODES, h), jnp.float32)],
    )(x, wl, wr, b.reshape(1, h))


def _mean_from_parts(agg_refs, cnt_ref):
    agg = jnp.concatenate([a[0] + a[1] for a in agg_refs], axis=1)
    cnt = cnt_ref[0, :, 0:1] + cnt_ref[1, :, 0:1]
    return agg / jnp.maximum(cnt, 1.0)


def _tc_mid_body(nparts, *refs):
    agg_refs = refs[:nparts]
    cnt_ref, z_ref, wl_ref, wr_ref, b_ref = refs[nparts:nparts + 5]
    out_refs = refs[nparts + 5:]
    h = jnp.maximum(_mean_from_parts(agg_refs, cnt_ref) + z_ref[...], 0.0)
    y = jnp.dot(h, wl_ref[...], preferred_element_type=jnp.float32)
    for p, yr in enumerate(out_refs[:-1]):
        yr[...] = y[:, p * AW:(p + 1) * AW]
    out_refs[-1][...] = (
        jnp.dot(h, wr_ref[...], preferred_element_type=jnp.float32) + b_ref[...])


def _tc_mid(agg_parts, cnt, z, wl, wr, b):
    d, h = wl.shape
    nparts = len(agg_parts)
    oparts = h // AW
    grid = (N_NODES // _BR,)
    return pl.pallas_call(
        functools.partial(_tc_mid_body, nparts),
        grid=grid,
        in_specs=[pl.BlockSpec((NC, _BR, AW), lambda i: (0, i, 0))] * nparts + [
            pl.BlockSpec((NC, _BR, CNTW), lambda i: (0, i, 0)),
            pl.BlockSpec((_BR, d), lambda i: (i, 0)),
            pl.BlockSpec((d, h), lambda i: (0, 0)),
            pl.BlockSpec((d, h), lambda i: (0, 0)),
            pl.BlockSpec((1, h), lambda i: (0, 0)),
        ],
        out_specs=[pl.BlockSpec((_BR, AW), lambda i: (i, 0))] * oparts
                  + [pl.BlockSpec((_BR, h), lambda i: (i, 0))],
        out_shape=[jax.ShapeDtypeStruct((N_NODES, AW), jnp.float32)] * oparts
                  + [jax.ShapeDtypeStruct((N_NODES, h), jnp.float32)],
    )(*agg_parts, cnt, z, wl, wr, b.reshape(1, h))


def _tc_out_body(agg_ref, cnt_ref, z_ref, o_ref):
    o_ref[...] = _mean_from_parts([agg_ref], cnt_ref) + z_ref[...]


def _tc_out(agg, cnt, z):
    h = z.shape[1]
    grid = (N_NODES // _BR,)
    return pl.pallas_call(
        _tc_out_body,
        grid=grid,
        in_specs=[
            pl.BlockSpec((NC, _BR, h), lambda i: (0, i, 0)),
            pl.BlockSpec((NC, _BR, CNTW), lambda i: (0, i, 0)),
            pl.BlockSpec((_BR, h), lambda i: (i, 0)),
        ],
        out_specs=pl.BlockSpec((_BR, h), lambda i: (i, 0)),
        out_shape=jax.ShapeDtypeStruct((N_NODES, h), jnp.float32),
    )(agg, cnt, z)


@jax.jit
def kernel(x, edge_index, Wl1, Wr1, b1, Wl2, Wr2, b2, Wl3, Wr3, b3):
    src = edge_index[0].astype(jnp.int32).reshape(NW, NCHUNK, CHUNK)
    dst = edge_index[1].astype(jnp.int32).reshape(NW, NCHUNK, CHUNK)

    cnt = _sc_cnt(dst)
    *y1, z1 = _tc_in(x, Wl1, Wr1, b1)
    agg1 = [_sc_agg64(yp, src, dst) for yp in y1]
    *y2, z2 = _tc_mid(agg1, cnt, z1, Wl2, Wr2, b2)
    agg2 = [_sc_agg64(yp, src, dst) for yp in y2]
    y3, z3 = _tc_mid(agg2, cnt, z2, Wl3, Wr3, b3)
    agg3 = _sc_agg64(y3, src, dst)
    return _tc_out(agg3, cnt, z3)


# speedup vs baseline: 9.1484x; 1.0783x over previous
import functools

import jax
import jax.numpy as jnp
from jax import lax
from jax.experimental import pallas as pl
from jax.experimental.pallas import tpu as pltpu
from jax.experimental.pallas import tpu_sc as plsc

N_NODES = 10000
N_EDGES = 320000
NC, NS = 2, 16
NW = NC * NS
EPW = N_EDGES // NW
CHUNK = 128
NCHUNKS_ALL = N_EDGES // CHUNK
NCHUNK = NCHUNKS_ALL // NW
NEXTRA = NCHUNKS_ALL - NCHUNK * NW
NPAD = 10112
ROWS_PER_SUB = NPAD // NS
ZROWS = 128
CNTW = 16

_mesh = plsc.VectorSubcoreMesh(core_axis_name="c", subcore_axis_name="s")


def _zero_vmem(ref, rows, width):
    lanes = width // 16

    def body(i, carry):
        ref[i // lanes, pl.ds((i % lanes) * 16, 16)] = jnp.zeros((16,), jnp.float32)
        return carry

    lax.fori_loop(0, rows * lanes, body, 0)


def _clear_rows(zbuf, shared, s):
    base = s * ROWS_PER_SUB
    for k in range(ROWS_PER_SUB // ZROWS):
        pltpu.sync_copy(zbuf, shared.at[pl.ds(base + k * ZROWS, ZROWS)])
    rem = ROWS_PER_SUB % ZROWS
    if rem:
        pltpu.sync_copy(zbuf.at[pl.ds(0, rem)],
                        shared.at[pl.ds(base + (ROWS_PER_SUB // ZROWS) * ZROWS, rem)])


def _make_sc_agg(width):
    scratch = [
        pltpu.VMEM((NCHUNK, CHUNK), jnp.int32),
        pltpu.VMEM((NCHUNK, CHUNK), jnp.int32),
        pltpu.VMEM((CHUNK,), jnp.int32),
        pltpu.VMEM((CHUNK,), jnp.int32),
        pltpu.VMEM((CHUNK, width), jnp.float32),
        pltpu.VMEM((CHUNK, width), jnp.float32),
        pltpu.VMEM((ZROWS, width), jnp.float32),
        pltpu.VMEM_SHARED((NPAD, width), jnp.float32),
        pltpu.SemaphoreType.DMA,
        pltpu.SemaphoreType.DMA,
        pltpu.SemaphoreType.DMA,
        pltpu.SemaphoreType.DMA,
    ]

    def body(y_hbm, src_hbm, dst_hbm, out_hbm, src_v, dst_v, xsrc_v, xdst_v,
             r0, r1, zbuf, acc, g0, g1, s0, s1):
        c = lax.axis_index("c")
        s = lax.axis_index("s")
        w = s * NC + c

        _zero_vmem(zbuf, ZROWS, width)
        _clear_rows(zbuf, acc, s)
        plsc.subcore_barrier()

        pltpu.sync_copy(src_hbm.at[pl.ds(w * NCHUNK, NCHUNK)], src_v)
        pltpu.sync_copy(dst_hbm.at[pl.ds(w * NCHUNK, NCHUNK)], dst_v)

        pltpu.async_copy(y_hbm.at[src_v.at[0]], r0, g0)
        pltpu.async_copy(y_hbm.at[src_v.at[1]], r1, g1)

        def step(t, carry):
            j = 2 * t
            pltpu.make_async_copy(y_hbm.at[src_v.at[j]], r0, g0).wait()
            pltpu.async_copy(r0, acc.at[dst_v.at[j]], s0, add=True)
            pltpu.make_async_copy(y_hbm.at[src_v.at[j + 1]], r1, g1).wait()
            pltpu.async_copy(r1, acc.at[dst_v.at[j + 1]], s1, add=True)
            pltpu.make_async_copy(r0, acc.at[dst_v.at[j]], s0).wait()
            pltpu.async_copy(y_hbm.at[src_v.at[j + 2]], r0, g0)
            pltpu.make_async_copy(r1, acc.at[dst_v.at[j + 1]], s1).wait()

            @pl.when(j + 3 < NCHUNK)
            def _():
                pltpu.async_copy(y_hbm.at[src_v.at[j + 3]], r1, g1)

            return carry

        lax.fori_loop(0, (NCHUNK - 1) // 2, step, 0)

        jl = NCHUNK - 2
        pltpu.make_async_copy(y_hbm.at[src_v.at[jl]], r0, g0).wait()
        pltpu.async_copy(r0, acc.at[dst_v.at[jl]], s0, add=True)
        pltpu.make_async_copy(y_hbm.at[src_v.at[jl + 1]], r1, g1).wait()
        pltpu.async_copy(r1, acc.at[dst_v.at[jl + 1]], s1, add=True)
        pltpu.make_async_copy(r0, acc.at[dst_v.at[jl]], s0).wait()
        pltpu.make_async_copy(r1, acc.at[dst_v.at[jl + 1]], s1).wait()

        @pl.when(w < NEXTRA)
        def _():
            pltpu.sync_copy(src_hbm.at[NCHUNK * NW + w], xsrc_v)
            pltpu.sync_copy(dst_hbm.at[NCHUNK * NW + w], xdst_v)
            pltpu.sync_copy(y_hbm.at[xsrc_v], r0)
            pltpu.sync_copy(r0, acc.at[xdst_v], add=True)

        plsc.subcore_barrier()

        rs = pl.ds(s * ROWS_PER_SUB, ROWS_PER_SUB)
        pltpu.sync_copy(acc.at[rs], out_hbm.at[c].at[rs])

    return pl.kernel(
        body,
        out_type=jax.ShapeDtypeStruct((NC, NPAD, width), jnp.float32),
        mesh=_mesh,
        scratch_types=scratch,
        compiler_params=pltpu.CompilerParams(use_tc_tiling_on_sc=False),
    )


_CNT_GROUP = 6


def _sc_cnt_body(dst_hbm, cnt_hbm, dst_v, xdst_v, ones_v, zbuf, cntacc, sem):
    c = lax.axis_index("c")
    s = lax.axis_index("s")
    w = s * NC + c

    _zero_vmem(zbuf, ZROWS, CNTW)
    _clear_rows(zbuf, cntacc, s)

    def ones_body(i, carry):
        ones_v[i, pl.ds(0, 16)] = jnp.ones((16,), jnp.float32)
        return carry

    lax.fori_loop(0, CHUNK, ones_body, 0)
    plsc.subcore_barrier()

    pltpu.sync_copy(dst_hbm.at[pl.ds(w * NCHUNK, NCHUNK)], dst_v)

    def step(t, carry):
        j = t * _CNT_GROUP
        for k in range(_CNT_GROUP):
            pltpu.async_copy(ones_v, cntacc.at[dst_v.at[j + k]], sem, add=True)
        for k in range(_CNT_GROUP):
            pltpu.make_async_copy(ones_v, cntacc.at[dst_v.at[j + k]], sem).wait()
        return carry

    lax.fori_loop(0, NCHUNK // _CNT_GROUP, step, 0)

    @pl.when(w < NEXTRA)
    def _():
        pltpu.sync_copy(dst_hbm.at[NCHUNK * NW + w], xdst_v)
        pltpu.sync_copy(ones_v, cntacc.at[xdst_v], add=True)

    plsc.subcore_barrier()

    rs = pl.ds(s * ROWS_PER_SUB, ROWS_PER_SUB)
    pltpu.sync_copy(cntacc.at[rs], cnt_hbm.at[c].at[rs])


_sc_cnt = pl.kernel(
    _sc_cnt_body,
    out_type=jax.ShapeDtypeStruct((NC, NPAD, CNTW), jnp.float32),
    mesh=_mesh,
    scratch_types=[
        pltpu.VMEM((NCHUNK, CHUNK), jnp.int32),
        pltpu.VMEM((CHUNK,), jnp.int32),
        pltpu.VMEM((CHUNK, CNTW), jnp.float32),
        pltpu.VMEM((ZROWS, CNTW), jnp.float32),
        pltpu.VMEM_SHARED((NPAD, CNTW), jnp.float32),
        pltpu.SemaphoreType.DMA,
    ],
    compiler_params=pltpu.CompilerParams(use_tc_tiling_on_sc=False),
)

_sc_agg64 = _make_sc_agg(64)


_BR = 1000
AW = 64


def _tc_in_body(x_ref, wl_ref, wr_ref, b_ref, *out_refs):
    x = x_ref[...]
    y = jnp.dot(x, wl_ref[...], preferred_element_type=jnp.float32)
    for p, yr in enumerate(out_refs[:-1]):
        yr[...] = y[:, p * AW:(p + 1) * AW]
    out_refs[-1][...] = (
        jnp.dot(x, wr_ref[...], preferred_element_type=jnp.float32) + b_ref[...])


def _tc_in(x, wl, wr, b):
    d, h = wl.shape
    parts = h // AW
    grid = (N_NODES // _BR,)
    return pl.pallas_call(
        _tc_in_body,
        grid=grid,
        in_specs=[
            pl.BlockSpec((_BR, d), lambda i: (i, 0)),
            pl.BlockSpec((d, h), lambda i: (0, 0)),
            pl.BlockSpec((d, h), lambda i: (0, 0)),
            pl.BlockSpec((1, h), lambda i: (0, 0)),
        ],
        out_specs=[pl.BlockSpec((_BR, AW), lambda i: (i, 0))] * parts
                  + [pl.BlockSpec((_BR, h), lambda i: (i, 0))],
        out_shape=[jax.ShapeDtypeStruct((N_NODES, AW), jnp.float32)] * parts
                  + [jax.ShapeDtypeStruct((N_NODES, h), jnp.float32)],
    )(x, wl, wr, b.reshape(1, h))


def _mean_from_parts(agg_refs, cnt_ref):
    agg = jnp.concatenate([a[0] + a[1] for a in agg_refs], axis=1)
    cnt = cnt_ref[0, :, 0:1] + cnt_ref[1, :, 0:1]
    return agg / jnp.maximum(cnt, 1.0)


def _tc_mid_body(nparts, *refs):
    agg_refs = refs[:nparts]
    cnt_ref, z_ref, wl_ref, wr_ref, b_ref = refs[nparts:nparts + 5]
    out_refs = refs[nparts + 5:]
    h = jnp.maximum(_mean_from_parts(agg_refs, cnt_ref) + z_ref[...], 0.0)
    y = jnp.dot(h, wl_ref[...], preferred_element_type=jnp.float32)
    for p, yr in enumerate(out_refs[:-1]):
        yr[...] = y[:, p * AW:(p + 1) * AW]
    out_refs[-1][...] = (
        jnp.dot(h, wr_ref[...], preferred_element_type=jnp.float32) + b_ref[...])


def _tc_mid(agg_parts, cnt, z, wl, wr, b):
    d, h = wl.shape
    nparts = len(agg_parts)
    oparts = h // AW
    grid = (N_NODES // _BR,)
    return pl.pallas_call(
        functools.partial(_tc_mid_body, nparts),
        grid=grid,
        in_specs=[pl.BlockSpec((NC, _BR, AW), lambda i: (0, i, 0))] * nparts + [
            pl.BlockSpec((NC, _BR, CNTW), lambda i: (0, i, 0)),
            pl.BlockSpec((_BR, d), lambda i: (i, 0)),
            pl.BlockSpec((d, h), lambda i: (0, 0)),
            pl.BlockSpec((d, h), lambda i: (0, 0)),
            pl.BlockSpec((1, h), lambda i: (0, 0)),
        ],
        out_specs=[pl.BlockSpec((_BR, AW), lambda i: (i, 0))] * oparts
                  + [pl.BlockSpec((_BR, h), lambda i: (i, 0))],
        out_shape=[jax.ShapeDtypeStruct((N_NODES, AW), jnp.float32)] * oparts
                  + [jax.ShapeDtypeStruct((N_NODES, h), jnp.float32)],
    )(*agg_parts, cnt, z, wl, wr, b.reshape(1, h))


def _tc_out_body(agg_ref, cnt_ref, z_ref, o_ref):
    o_ref[...] = _mean_from_parts([agg_ref], cnt_ref) + z_ref[...]


def _tc_out(agg, cnt, z):
    h = z.shape[1]
    grid = (N_NODES // _BR,)
    return pl.pallas_call(
        _tc_out_body,
        grid=grid,
        in_specs=[
            pl.BlockSpec((NC, _BR, h), lambda i: (0, i, 0)),
            pl.BlockSpec((NC, _BR, CNTW), lambda i: (0, i, 0)),
            pl.BlockSpec((_BR, h), lambda i: (i, 0)),
        ],
        out_specs=pl.BlockSpec((_BR, h), lambda i: (i, 0)),
        out_shape=jax.ShapeDtypeStruct((N_NODES, h), jnp.float32),
    )(agg, cnt, z)


@jax.jit
def kernel(x, edge_index, Wl1, Wr1, b1, Wl2, Wr2, b2, Wl3, Wr3, b3):
    src = edge_index[0].astype(jnp.int32).reshape(NCHUNKS_ALL, CHUNK)
    dst = edge_index[1].astype(jnp.int32).reshape(NCHUNKS_ALL, CHUNK)

    cnt = _sc_cnt(dst)
    *y1, z1 = _tc_in(x, Wl1, Wr1, b1)
    agg1 = [_sc_agg64(yp, src, dst) for yp in y1]
    *y2, z2 = _tc_mid(agg1, cnt, z1, Wl2, Wr2, b2)
    agg2 = [_sc_agg64(yp, src, dst) for yp in y2]
    y3, z3 = _tc_mid(agg2, cnt, z2, Wl3, Wr3, b3)
    agg3 = _sc_agg64(y3, src, dst)
    return _tc_out(agg3, cnt, z3)


# speedup vs baseline: 11.0434x; 1.2071x over previous
import functools

import jax
import jax.numpy as jnp
from jax import lax
from jax.experimental import pallas as pl
from jax.experimental.pallas import tpu as pltpu
from jax.experimental.pallas import tpu_sc as plsc

N_NODES = 10000
N_EDGES = 320000
NC, NS = 2, 16
NW = NC * NS
EPW = N_EDGES // NW
CHUNK = 128
NCHUNKS_ALL = N_EDGES // CHUNK
NCHUNK = NCHUNKS_ALL // NW
NEXTRA = NCHUNKS_ALL - NCHUNK * NW
NPAD = 10112
ROWS_PER_SUB = NPAD // NS
ZROWS = 128
CNTW = 16

_mesh = plsc.VectorSubcoreMesh(core_axis_name="c", subcore_axis_name="s")


def _zero_vmem(ref, rows, width):
    lanes = width // 16

    def body(i, carry):
        ref[i // lanes, pl.ds((i % lanes) * 16, 16)] = jnp.zeros((16,), jnp.float32)
        return carry

    lax.fori_loop(0, rows * lanes, body, 0)


def _clear_rows(zbuf, shared, s):
    base = s * ROWS_PER_SUB
    for k in range(ROWS_PER_SUB // ZROWS):
        pltpu.sync_copy(zbuf, shared.at[pl.ds(base + k * ZROWS, ZROWS)])
    rem = ROWS_PER_SUB % ZROWS
    if rem:
        pltpu.sync_copy(zbuf.at[pl.ds(0, rem)],
                        shared.at[pl.ds(base + (ROWS_PER_SUB // ZROWS) * ZROWS, rem)])


def _make_sc_agg(width, with_cnt=False):
    out_types = [jax.ShapeDtypeStruct((NC, NPAD, width), jnp.float32)]
    scratch = [
        pltpu.VMEM((NCHUNK, CHUNK), jnp.int32),
        pltpu.VMEM((NCHUNK, CHUNK), jnp.int32),
        pltpu.VMEM((CHUNK,), jnp.int32),
        pltpu.VMEM((CHUNK,), jnp.int32),
        pltpu.VMEM((CHUNK, width), jnp.float32),
        pltpu.VMEM((CHUNK, width), jnp.float32),
        pltpu.VMEM((CHUNK, width), jnp.float32),
        pltpu.VMEM((ZROWS, width), jnp.float32),
        pltpu.VMEM_SHARED((NPAD, width), jnp.float32),
        pltpu.SemaphoreType.DMA,
        pltpu.SemaphoreType.DMA,
        pltpu.SemaphoreType.DMA,
        pltpu.SemaphoreType.DMA,
        pltpu.SemaphoreType.DMA,
        pltpu.SemaphoreType.DMA,
    ]
    if with_cnt:
        out_types.append(jax.ShapeDtypeStruct((NC, NPAD, CNTW), jnp.float32))
        scratch += [
            pltpu.VMEM((CHUNK, CNTW), jnp.float32),
            pltpu.VMEM((ZROWS, CNTW), jnp.float32),
            pltpu.VMEM_SHARED((NPAD, CNTW), jnp.float32),
            pltpu.SemaphoreType.DMA,
        ]

    def body(y_hbm, src_hbm, dst_hbm, *rest):
        if with_cnt:
            (out_hbm, cnt_hbm, src_v, dst_v, xsrc_v, xdst_v, r0, r1, r2, zbuf,
             acc, g0, g1, g2, s0, s1, s2, ones_v, zcnt, cntacc, csem) = rest
        else:
            (out_hbm, src_v, dst_v, xsrc_v, xdst_v, r0, r1, r2, zbuf,
             acc, g0, g1, g2, s0, s1, s2) = rest
        c = lax.axis_index("c")
        s = lax.axis_index("s")
        w = s * NC + c
        bufs = ((r0, g0, s0), (r1, g1, s1), (r2, g2, s2))

        _zero_vmem(zbuf, ZROWS, width)
        _clear_rows(zbuf, acc, s)
        if with_cnt:
            _zero_vmem(zcnt, ZROWS, CNTW)
            _clear_rows(zcnt, cntacc, s)

            def ones_body(i, carry):
                ones_v[i, pl.ds(0, CNTW)] = jnp.ones((CNTW,), jnp.float32)
                return carry

            lax.fori_loop(0, CHUNK, ones_body, 0)
        plsc.subcore_barrier()

        pltpu.sync_copy(src_hbm.at[pl.ds(w * NCHUNK, NCHUNK)], src_v)
        pltpu.sync_copy(dst_hbm.at[pl.ds(w * NCHUNK, NCHUNK)], dst_v)

        for b, (rb, gb, sb) in enumerate(bufs):
            pltpu.async_copy(y_hbm.at[src_v.at[b]], rb, gb)

        def consume(jj, rb, gb, sb):
            pltpu.make_async_copy(y_hbm.at[src_v.at[jj]], rb, gb).wait()
            pltpu.async_copy(rb, acc.at[dst_v.at[jj]], sb, add=True)
            if with_cnt:
                pltpu.async_copy(ones_v, cntacc.at[dst_v.at[jj]], csem, add=True)

        def step(t, carry):
            j = 3 * t
            for b, (rb, gb, sb) in enumerate(bufs):
                consume(j + b, rb, gb, sb)
            for b, (rb, gb, sb) in enumerate(bufs):
                pltpu.make_async_copy(rb, acc.at[dst_v.at[j + b]], sb).wait()
                pltpu.async_copy(y_hbm.at[src_v.at[j + b + 3]], rb, gb)
            if with_cnt:
                for b in range(3):
                    pltpu.make_async_copy(
                        ones_v, cntacc.at[dst_v.at[j + b]], csem).wait()
            return carry

        lax.fori_loop(0, NCHUNK // 3 - 1, step, 0)

        jl = NCHUNK - 3
        for b, (rb, gb, sb) in enumerate(bufs):
            consume(jl + b, rb, gb, sb)
        for b, (rb, gb, sb) in enumerate(bufs):
            pltpu.make_async_copy(rb, acc.at[dst_v.at[jl + b]], sb).wait()
        if with_cnt:
            for b in range(3):
                pltpu.make_async_copy(
                    ones_v, cntacc.at[dst_v.at[jl + b]], csem).wait()

        @pl.when(w < NEXTRA)
        def _():
            pltpu.sync_copy(src_hbm.at[NCHUNK * NW + w], xsrc_v)
            pltpu.sync_copy(dst_hbm.at[NCHUNK * NW + w], xdst_v)
            pltpu.sync_copy(y_hbm.at[xsrc_v], r0)
            pltpu.sync_copy(r0, acc.at[xdst_v], add=True)
            if with_cnt:
                pltpu.sync_copy(ones_v, cntacc.at[xdst_v], add=True)

        plsc.subcore_barrier()

        rs = pl.ds(s * ROWS_PER_SUB, ROWS_PER_SUB)
        pltpu.sync_copy(acc.at[rs], out_hbm.at[c].at[rs])
        if with_cnt:
            pltpu.sync_copy(cntacc.at[rs], cnt_hbm.at[c].at[rs])

    return pl.kernel(
        body,
        out_type=tuple(out_types) if with_cnt else out_types[0],
        mesh=_mesh,
        scratch_types=scratch,
        compiler_params=pltpu.CompilerParams(use_tc_tiling_on_sc=False),
    )


_sc_agg64 = _make_sc_agg(64)
_sc_agg64_cnt = _make_sc_agg(64, with_cnt=True)


_BR = 1000
AW = 64


def _tc_in_body(x_ref, wl_ref, wr_ref, b_ref, *out_refs):
    x = x_ref[...]
    y = jnp.dot(x, wl_ref[...], preferred_element_type=jnp.float32)
    for p, yr in enumerate(out_refs[:-1]):
        yr[...] = y[:, p * AW:(p + 1) * AW]
    out_refs[-1][...] = (
        jnp.dot(x, wr_ref[...], preferred_element_type=jnp.float32) + b_ref[...])


def _tc_in(x, wl, wr, b):
    d, h = wl.shape
    parts = h // AW
    grid = (N_NODES // _BR,)
    return pl.pallas_call(
        _tc_in_body,
        grid=grid,
        in_specs=[
            pl.BlockSpec((_BR, d), lambda i: (i, 0)),
            pl.BlockSpec((d, h), lambda i: (0, 0)),
            pl.BlockSpec((d, h), lambda i: (0, 0)),
            pl.BlockSpec((1, h), lambda i: (0, 0)),
        ],
        out_specs=[pl.BlockSpec((_BR, AW), lambda i: (i, 0))] * parts
                  + [pl.BlockSpec((_BR, h), lambda i: (i, 0))],
        out_shape=[jax.ShapeDtypeStruct((N_NODES, AW), jnp.float32)] * parts
                  + [jax.ShapeDtypeStruct((N_NODES, h), jnp.float32)],
    )(x, wl, wr, b.reshape(1, h))


def _mean_from_parts(agg_refs, cnt_ref):
    agg = jnp.concatenate([a[0] + a[1] for a in agg_refs], axis=1)
    cnt = cnt_ref[0, :, 0:1] + cnt_ref[1, :, 0:1]
    return agg / jnp.maximum(cnt, 1.0)


def _tc_mid_body(nparts, *refs):
    agg_refs = refs[:nparts]
    cnt_ref, z_ref, wl_ref, wr_ref, b_ref = refs[nparts:nparts + 5]
    out_refs = refs[nparts + 5:]
    h = jnp.maximum(_mean_from_parts(agg_refs, cnt_ref) + z_ref[...], 0.0)
    y = jnp.dot(h, wl_ref[...], preferred_element_type=jnp.float32)
    for p, yr in enumerate(out_refs[:-1]):
        yr[...] = y[:, p * AW:(p + 1) * AW]
    out_refs[-1][...] = (
        jnp.dot(h, wr_ref[...], preferred_element_type=jnp.float32) + b_ref[...])


def _tc_mid(agg_parts, cnt, z, wl, wr, b):
    d, h = wl.shape
    nparts = len(agg_parts)
    oparts = h // AW
    grid = (N_NODES // _BR,)
    return pl.pallas_call(
        functools.partial(_tc_mid_body, nparts),
        grid=grid,
        in_specs=[pl.BlockSpec((NC, _BR, AW), lambda i: (0, i, 0))] * nparts + [
            pl.BlockSpec((NC, _BR, CNTW), lambda i: (0, i, 0)),
            pl.BlockSpec((_BR, d), lambda i: (i, 0)),
            pl.BlockSpec((d, h), lambda i: (0, 0)),
            pl.BlockSpec((d, h), lambda i: (0, 0)),
            pl.BlockSpec((1, h), lambda i: (0, 0)),
        ],
        out_specs=[pl.BlockSpec((_BR, AW), lambda i: (i, 0))] * oparts
                  + [pl.BlockSpec((_BR, h), lambda i: (i, 0))],
        out_shape=[jax.ShapeDtypeStruct((N_NODES, AW), jnp.float32)] * oparts
                  + [jax.ShapeDtypeStruct((N_NODES, h), jnp.float32)],
    )(*agg_parts, cnt, z, wl, wr, b.reshape(1, h))


def _tc_out_body(agg_ref, cnt_ref, z_ref, o_ref):
    o_ref[...] = _mean_from_parts([agg_ref], cnt_ref) + z_ref[...]


def _tc_out(agg, cnt, z):
    h = z.shape[1]
    grid = (N_NODES // _BR,)
    return pl.pallas_call(
        _tc_out_body,
        grid=grid,
        in_specs=[
            pl.BlockSpec((NC, _BR, h), lambda i: (0, i, 0)),
            pl.BlockSpec((NC, _BR, CNTW), lambda i: (0, i, 0)),
            pl.BlockSpec((_BR, h), lambda i: (i, 0)),
        ],
        out_specs=pl.BlockSpec((_BR, h), lambda i: (i, 0)),
        out_shape=jax.ShapeDtypeStruct((N_NODES, h), jnp.float32),
    )(agg, cnt, z)


@jax.jit
def kernel(x, edge_index, Wl1, Wr1, b1, Wl2, Wr2, b2, Wl3, Wr3, b3):
    src = edge_index[0].astype(jnp.int32).reshape(NCHUNKS_ALL, CHUNK)
    dst = edge_index[1].astype(jnp.int32).reshape(NCHUNKS_ALL, CHUNK)

    *y1, z1 = _tc_in(x, Wl1, Wr1, b1)
    agg1_lo, cnt = _sc_agg64_cnt(y1[0], src, dst)
    agg1 = [agg1_lo] + [_sc_agg64(yp, src, dst) for yp in y1[1:]]
    *y2, z2 = _tc_mid(agg1, cnt, z1, Wl2, Wr2, b2)
    agg2 = [_sc_agg64(yp, src, dst) for yp in y2]
    y3, z3 = _tc_mid(agg2, cnt, z2, Wl3, Wr3, b3)
    agg3 = _sc_agg64(y3, src, dst)
    return _tc_out(agg3, cnt, z3)


# speedup vs baseline: 11.3739x; 1.0299x over previous
import functools

import jax
import jax.numpy as jnp
from jax import lax
from jax.experimental import pallas as pl
from jax.experimental.pallas import tpu as pltpu
from jax.experimental.pallas import tpu_sc as plsc

N_NODES = 10000
N_EDGES = 320000
NC, NS = 2, 16
NW = NC * NS
EPW = N_EDGES // NW
CHUNK = 125
NCHUNKS_ALL = N_EDGES // CHUNK
NCHUNK = NCHUNKS_ALL // NW
NPAD = 10112
ROWS_PER_SUB = NPAD // NS
ZROWS = 128
CNTW = 16

_mesh = plsc.VectorSubcoreMesh(core_axis_name="c", subcore_axis_name="s")


def _zero_vmem(ref, rows, width):
    lanes = width // 16

    def body(i, carry):
        ref[i // lanes, pl.ds((i % lanes) * 16, 16)] = jnp.zeros((16,), jnp.float32)
        return carry

    lax.fori_loop(0, rows * lanes, body, 0)


def _clear_rows(zbuf, shared, s):
    base = s * ROWS_PER_SUB
    for k in range(ROWS_PER_SUB // ZROWS):
        pltpu.sync_copy(zbuf, shared.at[pl.ds(base + k * ZROWS, ZROWS)])
    rem = ROWS_PER_SUB % ZROWS
    if rem:
        pltpu.sync_copy(zbuf.at[pl.ds(0, rem)],
                        shared.at[pl.ds(base + (ROWS_PER_SUB // ZROWS) * ZROWS, rem)])


def _make_sc_agg(width, with_cnt=False, nbuf=4):
    out_types = [jax.ShapeDtypeStruct((NC, NPAD, width), jnp.float32)]
    scratch = (
        [
            pltpu.VMEM((NCHUNK, CHUNK), jnp.int32),
            pltpu.VMEM((NCHUNK, CHUNK), jnp.int32),
        ]
        + [pltpu.VMEM((CHUNK, width), jnp.float32)] * nbuf
        + [
            pltpu.VMEM((ZROWS, width), jnp.float32),
            pltpu.VMEM_SHARED((NPAD, width), jnp.float32),
        ]
        + [pltpu.SemaphoreType.DMA] * (2 * nbuf)
    )
    if with_cnt:
        out_types.append(jax.ShapeDtypeStruct((NC, NPAD, CNTW), jnp.float32))
        scratch += [
            pltpu.VMEM((CHUNK, CNTW), jnp.float32),
            pltpu.VMEM((ZROWS, CNTW), jnp.float32),
            pltpu.VMEM_SHARED((NPAD, CNTW), jnp.float32),
            pltpu.SemaphoreType.DMA,
        ]

    def body(y_hbm, src_hbm, dst_hbm, *rest):
        if with_cnt:
            out_hbm, cnt_hbm = rest[0], rest[1]
            rest = rest[2:]
        else:
            out_hbm = rest[0]
            rest = rest[1:]
        src_v, dst_v = rest[0], rest[1]
        rows = rest[2:2 + nbuf]
        zbuf, acc = rest[2 + nbuf], rest[3 + nbuf]
        gsems = rest[4 + nbuf:4 + 2 * nbuf]
        ssems = rest[4 + 2 * nbuf:4 + 3 * nbuf]
        if with_cnt:
            ones_v, zcnt, cntacc, csem = rest[4 + 3 * nbuf:]
        c = lax.axis_index("c")
        s = lax.axis_index("s")
        w = s * NC + c
        bufs = tuple(zip(rows, gsems, ssems))

        _zero_vmem(zbuf, ZROWS, width)
        _clear_rows(zbuf, acc, s)
        if with_cnt:
            _zero_vmem(zcnt, ZROWS, CNTW)
            _clear_rows(zcnt, cntacc, s)

            def ones_body(i, carry):
                ones_v[i, pl.ds(0, CNTW)] = jnp.ones((CNTW,), jnp.float32)
                return carry

            lax.fori_loop(0, CHUNK, ones_body, 0)
        plsc.subcore_barrier()

        pltpu.sync_copy(src_hbm.at[pl.ds(w * NCHUNK, NCHUNK)], src_v)
        pltpu.sync_copy(dst_hbm.at[pl.ds(w * NCHUNK, NCHUNK)], dst_v)

        for b, (rb, gb, sb) in enumerate(bufs):
            pltpu.async_copy(y_hbm.at[src_v.at[b]], rb, gb)

        def consume(jj, rb, gb, sb):
            pltpu.make_async_copy(y_hbm.at[src_v.at[jj]], rb, gb).wait()
            pltpu.async_copy(rb, acc.at[dst_v.at[jj]], sb, add=True)
            if with_cnt:
                pltpu.async_copy(ones_v, cntacc.at[dst_v.at[jj]], csem, add=True)

        def step(t, carry):
            j = nbuf * t
            for b, (rb, gb, sb) in enumerate(bufs):
                consume(j + b, rb, gb, sb)
            for b, (rb, gb, sb) in enumerate(bufs):
                pltpu.make_async_copy(rb, acc.at[dst_v.at[j + b]], sb).wait()
                pltpu.async_copy(y_hbm.at[src_v.at[j + b + nbuf]], rb, gb)
            if with_cnt:
                for b in range(nbuf):
                    pltpu.make_async_copy(
                        ones_v, cntacc.at[dst_v.at[j + b]], csem).wait()
            return carry

        lax.fori_loop(0, NCHUNK // nbuf - 1, step, 0)

        jl = NCHUNK - nbuf
        for b, (rb, gb, sb) in enumerate(bufs):
            consume(jl + b, rb, gb, sb)
        for b, (rb, gb, sb) in enumerate(bufs):
            pltpu.make_async_copy(rb, acc.at[dst_v.at[jl + b]], sb).wait()
        if with_cnt:
            for b in range(nbuf):
                pltpu.make_async_copy(
                    ones_v, cntacc.at[dst_v.at[jl + b]], csem).wait()

        plsc.subcore_barrier()

        rs = pl.ds(s * ROWS_PER_SUB, ROWS_PER_SUB)
        pltpu.sync_copy(acc.at[rs], out_hbm.at[c].at[rs])
        if with_cnt:
            pltpu.sync_copy(cntacc.at[rs], cnt_hbm.at[c].at[rs])

    return pl.kernel(
        body,
        out_type=tuple(out_types) if with_cnt else out_types[0],
        mesh=_mesh,
        scratch_types=scratch,
        compiler_params=pltpu.CompilerParams(use_tc_tiling_on_sc=False),
    )


_sc_agg64 = _make_sc_agg(64, nbuf=4)
_sc_agg64_cnt = _make_sc_agg(64, with_cnt=True, nbuf=2)


_BR = 1000
AW = 64


def _tc_in_body(x_ref, wl_ref, wr_ref, b_ref, *out_refs):
    x = x_ref[...]
    y = jnp.dot(x, wl_ref[...], preferred_element_type=jnp.float32)
    for p, yr in enumerate(out_refs[:-1]):
        yr[...] = y[:, p * AW:(p + 1) * AW]
    out_refs[-1][...] = (
        jnp.dot(x, wr_ref[...], preferred_element_type=jnp.float32) + b_ref[...])


def _tc_in(x, wl, wr, b):
    d, h = wl.shape
    parts = h // AW
    grid = (N_NODES // _BR,)
    return pl.pallas_call(
        _tc_in_body,
        grid=grid,
        in_specs=[
            pl.BlockSpec((_BR, d), lambda i: (i, 0)),
            pl.BlockSpec((d, h), lambda i: (0, 0)),
            pl.BlockSpec((d, h), lambda i: (0, 0)),
            pl.BlockSpec((1, h), lambda i: (0, 0)),
        ],
        out_specs=[pl.BlockSpec((_BR, AW), lambda i: (i, 0))] * parts
                  + [pl.BlockSpec((_BR, h), lambda i: (i, 0))],
        out_shape=[jax.ShapeDtypeStruct((N_NODES, AW), jnp.float32)] * parts
                  + [jax.ShapeDtypeStruct((N_NODES, h), jnp.float32)],
    )(x, wl, wr, b.reshape(1, h))


def _mean_from_parts(agg_refs, cnt_ref):
    agg = jnp.concatenate([a[0] + a[1] for a in agg_refs], axis=1)
    cnt = cnt_ref[0, :, 0:1] + cnt_ref[1, :, 0:1]
    return agg / jnp.maximum(cnt, 1.0)


def _tc_mid_body(nparts, *refs):
    agg_refs = refs[:nparts]
    cnt_ref, z_ref, wl_ref, wr_ref, b_ref = refs[nparts:nparts + 5]
    out_refs = refs[nparts + 5:]
    h = jnp.maximum(_mean_from_parts(agg_refs, cnt_ref) + z_ref[...], 0.0)
    y = jnp.dot(h, wl_ref[...], preferred_element_type=jnp.float32)
    for p, yr in enumerate(out_refs[:-1]):
        yr[...] = y[:, p * AW:(p + 1) * AW]
    out_refs[-1][...] = (
        jnp.dot(h, wr_ref[...], preferred_element_type=jnp.float32) + b_ref[...])


def _tc_mid(agg_parts, cnt, z, wl, wr, b):
    d, h = wl.shape
    nparts = len(agg_parts)
    oparts = h // AW
    grid = (N_NODES // _BR,)
    return pl.pallas_call(
        functools.partial(_tc_mid_body, nparts),
        grid=grid,
        in_specs=[pl.BlockSpec((NC, _BR, AW), lambda i: (0, i, 0))] * nparts + [
            pl.BlockSpec((NC, _BR, CNTW), lambda i: (0, i, 0)),
            pl.BlockSpec((_BR, d), lambda i: (i, 0)),
            pl.BlockSpec((d, h), lambda i: (0, 0)),
            pl.BlockSpec((d, h), lambda i: (0, 0)),
            pl.BlockSpec((1, h), lambda i: (0, 0)),
        ],
        out_specs=[pl.BlockSpec((_BR, AW), lambda i: (i, 0))] * oparts
                  + [pl.BlockSpec((_BR, h), lambda i: (i, 0))],
        out_shape=[jax.ShapeDtypeStruct((N_NODES, AW), jnp.float32)] * oparts
                  + [jax.ShapeDtypeStruct((N_NODES, h), jnp.float32)],
    )(*agg_parts, cnt, z, wl, wr, b.reshape(1, h))


def _tc_out_body(agg_ref, cnt_ref, z_ref, o_ref):
    o_ref[...] = _mean_from_parts([agg_ref], cnt_ref) + z_ref[...]


def _tc_out(agg, cnt, z):
    h = z.shape[1]
    grid = (N_NODES // _BR,)
    return pl.pallas_call(
        _tc_out_body,
        grid=grid,
        in_specs=[
            pl.BlockSpec((NC, _BR, h), lambda i: (0, i, 0)),
            pl.BlockSpec((NC, _BR, CNTW), lambda i: (0, i, 0)),
            pl.BlockSpec((_BR, h), lambda i: (i, 0)),
        ],
        out_specs=pl.BlockSpec((_BR, h), lambda i: (i, 0)),
        out_shape=jax.ShapeDtypeStruct((N_NODES, h), jnp.float32),
    )(agg, cnt, z)


@jax.jit
def kernel(x, edge_index, Wl1, Wr1, b1, Wl2, Wr2, b2, Wl3, Wr3, b3):
    src = edge_index[0].astype(jnp.int32).reshape(NCHUNKS_ALL, CHUNK)
    dst = edge_index[1].astype(jnp.int32).reshape(NCHUNKS_ALL, CHUNK)

    *y1, z1 = _tc_in(x, Wl1, Wr1, b1)
    agg1_lo, cnt = _sc_agg64_cnt(y1[0], src, dst)
    agg1 = [agg1_lo] + [_sc_agg64(yp, src, dst) for yp in y1[1:]]
    *y2, z2 = _tc_mid(agg1, cnt, z1, Wl2, Wr2, b2)
    agg2 = [_sc_agg64(yp, src, dst) for yp in y2]
    y3, z3 = _tc_mid(agg2, cnt, z2, Wl3, Wr3, b3)
    agg3 = _sc_agg64(y3, src, dst)
    return _tc_out(agg3, cnt, z3)


# speedup vs baseline: 11.8316x; 1.0402x over previous
import functools

import jax
import jax.numpy as jnp
from jax import lax
from jax.experimental import pallas as pl
from jax.experimental.pallas import tpu as pltpu
from jax.experimental.pallas import tpu_sc as plsc

N_NODES = 10000
N_EDGES = 320000
NC, NS = 2, 16
NW = NC * NS
EPW = N_EDGES // NW
CHUNK = 125
NCHUNKS_ALL = N_EDGES // CHUNK
NCHUNK = NCHUNKS_ALL // NW
NPAD = 10112
ROWS_PER_SUB = NPAD // NS
ZROWS = 128
CNTW = 16

_mesh = plsc.VectorSubcoreMesh(core_axis_name="c", subcore_axis_name="s")


def _zero_vmem(ref, rows, width):
    lanes = width // 16

    def body(i, carry):
        ref[i // lanes, pl.ds((i % lanes) * 16, 16)] = jnp.zeros((16,), jnp.float32)
        return carry

    lax.fori_loop(0, rows * lanes, body, 0)


def _clear_rows(zbuf, shared, s):
    base = s * ROWS_PER_SUB
    for k in range(ROWS_PER_SUB // ZROWS):
        pltpu.sync_copy(zbuf, shared.at[pl.ds(base + k * ZROWS, ZROWS)])
    rem = ROWS_PER_SUB % ZROWS
    if rem:
        pltpu.sync_copy(zbuf.at[pl.ds(0, rem)],
                        shared.at[pl.ds(base + (ROWS_PER_SUB // ZROWS) * ZROWS, rem)])


def _make_sc_agg(width, with_cnt=False, nbuf=4):
    out_types = [jax.ShapeDtypeStruct((NC, NPAD, width), jnp.float32)]
    scratch = (
        [
            pltpu.VMEM((NCHUNK, CHUNK), jnp.int32),
            pltpu.VMEM((NCHUNK, CHUNK), jnp.int32),
        ]
        + [pltpu.VMEM((CHUNK, width), jnp.float32)] * nbuf
        + [
            pltpu.VMEM((ZROWS, width), jnp.float32),
            pltpu.VMEM_SHARED((NPAD, width), jnp.float32),
        ]
        + [pltpu.SemaphoreType.DMA] * (2 * nbuf)
    )
    if with_cnt:
        out_types.append(jax.ShapeDtypeStruct((NC, NPAD, CNTW), jnp.float32))
        scratch += [
            pltpu.VMEM((CHUNK, CNTW), jnp.float32),
            pltpu.VMEM((ZROWS, CNTW), jnp.float32),
            pltpu.VMEM_SHARED((NPAD, CNTW), jnp.float32),
            pltpu.SemaphoreType.DMA,
        ]

    def body(y_hbm, ei_hbm, *rest):
        if with_cnt:
            out_hbm, cnt_hbm = rest[0], rest[1]
            rest = rest[2:]
        else:
            out_hbm = rest[0]
            rest = rest[1:]
        src_v, dst_v = rest[0], rest[1]
        rows = rest[2:2 + nbuf]
        zbuf, acc = rest[2 + nbuf], rest[3 + nbuf]
        gsems = rest[4 + nbuf:4 + 2 * nbuf]
        ssems = rest[4 + 2 * nbuf:4 + 3 * nbuf]
        if with_cnt:
            ones_v, zcnt, cntacc, csem = rest[4 + 3 * nbuf:]
        c = lax.axis_index("c")
        s = lax.axis_index("s")
        w = s * NC + c
        bufs = tuple(zip(rows, gsems, ssems))

        _zero_vmem(zbuf, ZROWS, width)
        _clear_rows(zbuf, acc, s)
        if with_cnt:
            _zero_vmem(zcnt, ZROWS, CNTW)
            _clear_rows(zcnt, cntacc, s)

            def ones_body(i, carry):
                ones_v[i, pl.ds(0, CNTW)] = jnp.ones((CNTW,), jnp.float32)
                return carry

            lax.fori_loop(0, CHUNK, ones_body, 0)
        plsc.subcore_barrier()

        pltpu.sync_copy(ei_hbm.at[0].at[pl.ds(w * NCHUNK, NCHUNK)], src_v)
        pltpu.sync_copy(ei_hbm.at[1].at[pl.ds(w * NCHUNK, NCHUNK)], dst_v)

        for b, (rb, gb, sb) in enumerate(bufs):
            pltpu.async_copy(y_hbm.at[src_v.at[b]], rb, gb)

        def consume(jj, rb, gb, sb):
            pltpu.make_async_copy(y_hbm.at[src_v.at[jj]], rb, gb).wait()
            pltpu.async_copy(rb, acc.at[dst_v.at[jj]], sb, add=True)
            if with_cnt:
                pltpu.async_copy(ones_v, cntacc.at[dst_v.at[jj]], csem, add=True)

        def step(t, carry):
            j = nbuf * t
            for b, (rb, gb, sb) in enumerate(bufs):
                consume(j + b, rb, gb, sb)
            for b, (rb, gb, sb) in enumerate(bufs):
                pltpu.make_async_copy(rb, acc.at[dst_v.at[j + b]], sb).wait()
                pltpu.async_copy(y_hbm.at[src_v.at[j + b + nbuf]], rb, gb)
            if with_cnt:
                for b in range(nbuf):
                    pltpu.make_async_copy(
                        ones_v, cntacc.at[dst_v.at[j + b]], csem).wait()
            return carry

        lax.fori_loop(0, NCHUNK // nbuf - 1, step, 0)

        jl = NCHUNK - nbuf
        for b, (rb, gb, sb) in enumerate(bufs):
            consume(jl + b, rb, gb, sb)
        for b, (rb, gb, sb) in enumerate(bufs):
            pltpu.make_async_copy(rb, acc.at[dst_v.at[jl + b]], sb).wait()
        if with_cnt:
            for b in range(nbuf):
                pltpu.make_async_copy(
                    ones_v, cntacc.at[dst_v.at[jl + b]], csem).wait()

        plsc.subcore_barrier()

        rs = pl.ds(s * ROWS_PER_SUB, ROWS_PER_SUB)
        pltpu.sync_copy(acc.at[rs], out_hbm.at[c].at[rs])
        if with_cnt:
            pltpu.sync_copy(cntacc.at[rs], cnt_hbm.at[c].at[rs])

    return pl.kernel(
        body,
        out_type=tuple(out_types) if with_cnt else out_types[0],
        mesh=_mesh,
        scratch_types=scratch,
        compiler_params=pltpu.CompilerParams(use_tc_tiling_on_sc=False),
    )


_sc_agg64 = _make_sc_agg(64, nbuf=4)
_sc_agg64_cnt = _make_sc_agg(64, with_cnt=True, nbuf=2)


_BR = 2000
AW = 64


def _tc_in_body(x_ref, wl_ref, wr_ref, b_ref, *out_refs):
    x = x_ref[...]
    y = jnp.dot(x, wl_ref[...], preferred_element_type=jnp.float32)
    for p, yr in enumerate(out_refs[:-1]):
        yr[...] = y[:, p * AW:(p + 1) * AW]
    out_refs[-1][...] = (
        jnp.dot(x, wr_ref[...], preferred_element_type=jnp.float32) + b_ref[...])


def _tc_in(x, wl, wr, b):
    d, h = wl.shape
    parts = h // AW
    grid = (N_NODES // _BR,)
    return pl.pallas_call(
        _tc_in_body,
        grid=grid,
        in_specs=[
            pl.BlockSpec((_BR, d), lambda i: (i, 0)),
            pl.BlockSpec((d, h), lambda i: (0, 0)),
            pl.BlockSpec((d, h), lambda i: (0, 0)),
            pl.BlockSpec((1, h), lambda i: (0, 0)),
        ],
        out_specs=[pl.BlockSpec((_BR, AW), lambda i: (i, 0))] * parts
                  + [pl.BlockSpec((_BR, h), lambda i: (i, 0))],
        out_shape=[jax.ShapeDtypeStruct((N_NODES, AW), jnp.float32)] * parts
                  + [jax.ShapeDtypeStruct((N_NODES, h), jnp.float32)],
    )(x, wl, wr, b.reshape(1, h))


def _mean_from_parts(agg_refs, cnt_ref):
    agg = jnp.concatenate([a[0] + a[1] for a in agg_refs], axis=1)
    cnt = cnt_ref[0, :, 0:1] + cnt_ref[1, :, 0:1]
    return agg / jnp.maximum(cnt, 1.0)


def _tc_mid_body(nparts, *refs):
    agg_refs = refs[:nparts]
    cnt_ref, z_ref, wl_ref, wr_ref, b_ref = refs[nparts:nparts + 5]
    out_refs = refs[nparts + 5:]
    h = jnp.maximum(_mean_from_parts(agg_refs, cnt_ref) + z_ref[...], 0.0)
    y = jnp.dot(h, wl_ref[...], preferred_element_type=jnp.float32)
    for p, yr in enumerate(out_refs[:-1]):
        yr[...] = y[:, p * AW:(p + 1) * AW]
    out_refs[-1][...] = (
        jnp.dot(h, wr_ref[...], preferred_element_type=jnp.float32) + b_ref[...])


def _tc_mid(agg_parts, cnt, z, wl, wr, b):
    d, h = wl.shape
    nparts = len(agg_parts)
    oparts = h // AW
    grid = (N_NODES // _BR,)
    return pl.pallas_call(
        functools.partial(_tc_mid_body, nparts),
        grid=grid,
        in_specs=[pl.BlockSpec((NC, _BR, AW), lambda i: (0, i, 0))] * nparts + [
            pl.BlockSpec((NC, _BR, CNTW), lambda i: (0, i, 0)),
            pl.BlockSpec((_BR, d), lambda i: (i, 0)),
            pl.BlockSpec((d, h), lambda i: (0, 0)),
            pl.BlockSpec((d, h), lambda i: (0, 0)),
            pl.BlockSpec((1, h), lambda i: (0, 0)),
        ],
        out_specs=[pl.BlockSpec((_BR, AW), lambda i: (i, 0))] * oparts
                  + [pl.BlockSpec((_BR, h), lambda i: (i, 0))],
        out_shape=[jax.ShapeDtypeStruct((N_NODES, AW), jnp.float32)] * oparts
                  + [jax.ShapeDtypeStruct((N_NODES, h), jnp.float32)],
    )(*agg_parts, cnt, z, wl, wr, b.reshape(1, h))


def _tc_out_body(agg_ref, cnt_ref, z_ref, o_ref):
    o_ref[...] = _mean_from_parts([agg_ref], cnt_ref) + z_ref[...]


def _tc_out(agg, cnt, z):
    h = z.shape[1]
    grid = (N_NODES // _BR,)
    return pl.pallas_call(
        _tc_out_body,
        grid=grid,
        in_specs=[
            pl.BlockSpec((NC, _BR, h), lambda i: (0, i, 0)),
            pl.BlockSpec((NC, _BR, CNTW), lambda i: (0, i, 0)),
            pl.BlockSpec((_BR, h), lambda i: (i, 0)),
        ],
        out_specs=pl.BlockSpec((_BR, h), lambda i: (i, 0)),
        out_shape=jax.ShapeDtypeStruct((N_NODES, h), jnp.float32),
    )(agg, cnt, z)


@jax.jit
def kernel(x, edge_index, Wl1, Wr1, b1, Wl2, Wr2, b2, Wl3, Wr3, b3):
    ei = edge_index.astype(jnp.int32).reshape(2, NCHUNKS_ALL, CHUNK)

    *y1, z1 = _tc_in(x, Wl1, Wr1, b1)
    agg1_lo, cnt = _sc_agg64_cnt(y1[0], ei)
    agg1 = [agg1_lo] + [_sc_agg64(yp, ei) for yp in y1[1:]]
    *y2, z2 = _tc_mid(agg1, cnt, z1, Wl2, Wr2, b2)
    agg2 = [_sc_agg64(yp, ei) for yp in y2]
    y3, z3 = _tc_mid(agg2, cnt, z2, Wl3, Wr3, b3)
    agg3 = _sc_agg64(y3, ei)
    return _tc_out(agg3, cnt, z3)
